# baseline (speedup 1.0000x reference)
import functools

import jax
import jax.numpy as jnp
from jax.experimental import pallas as pl
from jax.experimental.pallas import tpu as pltpu


def _dense_moe_kernel(w_ref, xT_ref, wg_ref, wu_ref, wd_ref, out_ref):
    e = pl.program_id(1)

    xT = xT_ref[...].astype(jnp.bfloat16)
    g = jax.lax.dot_general(wg_ref[0], xT, (((1,), (0,)), ((), ())),
                            preferred_element_type=jnp.float32)
    u = jax.lax.dot_general(wu_ref[0], xT, (((1,), (0,)), ((), ())),
                            preferred_element_type=jnp.float32)
    h = (g * jax.nn.sigmoid(g) * u).astype(jnp.bfloat16)
    y = jax.lax.dot_general(wd_ref[0], h, (((1,), (0,)), ((), ())),
                            preferred_element_type=jnp.float32)
    y = y * w_ref[0, 0]

    @pl.when(e == 0)
    def _():
        out_ref[...] = y

    @pl.when(e > 0)
    def _():
        out_ref[...] += y


def kernel(x, Wgate, Wg, Wu, Wd, Wg_s, Wu_s, Wd_s):
    bsz, seq_len, H = x.shape
    E, I, _ = Wg.shape
    T = bsz * seq_len
    TOP_K = 2
    NE = E + 1

    xf = x.reshape(T, H)

    logits = xf @ Wgate.T
    scores = jax.nn.softmax(logits, axis=-1)
    topk_w, topk_idx = jax.lax.top_k(scores, TOP_K)
    denom = jnp.sum(topk_w, axis=-1, keepdims=True) + 1e-20
    topk_w = topk_w / denom

    w9 = jnp.zeros((T, NE), jnp.float32)
    w9 = w9.at[jnp.arange(T)[:, None], topk_idx].set(topk_w)
    w9 = w9.at[:, E].set(1.0)

    TM = min(512, T)
    n_tiles = T // TM
    warr = w9.T.reshape(NE, n_tiles, 1, TM)

    Wg_all = jnp.concatenate([Wg, Wg_s[None]], axis=0).astype(jnp.bfloat16)
    Wu_all = jnp.concatenate([Wu, Wu_s[None]], axis=0).astype(jnp.bfloat16)
    Wd_all = jnp.concatenate([Wd, Wd_s[None]], axis=0).astype(jnp.bfloat16)

    xT = xf.T

    grid = (n_tiles, NE)
    yT = pl.pallas_call(
        _dense_moe_kernel,
        grid=grid,
        in_specs=[
            pl.BlockSpec((1, 1, 1, TM), lambda t, e: (e, t, 0, 0)),
            pl.BlockSpec((H, TM), lambda t, e: (0, t)),
            pl.BlockSpec((1, I, H), lambda t, e: (e, 0, 0)),
            pl.BlockSpec((1, I, H), lambda t, e: (e, 0, 0)),
            pl.BlockSpec((1, H, I), lambda t, e: (e, 0, 0)),
        ],
        out_specs=pl.BlockSpec((H, TM), lambda t, e: (0, t)),
        out_shape=jax.ShapeDtypeStruct((H, T), jnp.float32),
    )(warr, xT, Wg_all, Wu_all, Wd_all)

    return yT.T.reshape(bsz, seq_len, H)


# speedup vs baseline: 1.0181x; 1.0181x over previous
import functools

import jax
import jax.numpy as jnp
from jax.experimental import pallas as pl
from jax.experimental.pallas import tpu as pltpu

TOP_K = 2


def _grouped_ffn_kernel(gid_ref, act_ref, xs_ref, ws_ref, wg_ref, wu_ref,
                        wd_ref, prev_ref, out_ref, *, first, k, I2):
    t = pl.program_id(0)

    @pl.when(act_ref[t] == 1)
    def _():
        x = xs_ref[...]
        g = jax.lax.dot_general(x, wg_ref[0, 0], (((1,), (1,)), ((), ())),
                                preferred_element_type=jnp.float32)
        u = jax.lax.dot_general(x, wu_ref[0, 0], (((1,), (1,)), ((), ())),
                                preferred_element_type=jnp.float32)
        h = g * jax.nn.sigmoid(g) * u
        wd = wd_ref[0, :, k * I2:(k + 1) * I2]
        y = jax.lax.dot_general(h, wd, (((1,), (1,)), ((), ())),
                                preferred_element_type=jnp.float32)
        y = y * ws_ref[0]
        if first:
            out_ref[...] = y
        else:
            out_ref[...] = prev_ref[...] + y


def _shared_ffn_kernel(x_ref, wg_ref, wu_ref, wd_ref, out_ref):
    x = x_ref[...].astype(jnp.bfloat16)
    g = jax.lax.dot_general(x, wg_ref[...], (((1,), (1,)), ((), ())),
                            preferred_element_type=jnp.float32)
    u = jax.lax.dot_general(x, wu_ref[...], (((1,), (1,)), ((), ())),
                            preferred_element_type=jnp.float32)
    h = (g * jax.nn.sigmoid(g) * u).astype(jnp.bfloat16)
    out_ref[...] = jax.lax.dot_general(h, wd_ref[...], (((1,), (1,)), ((), ())),
                                       preferred_element_type=jnp.float32)


def _routed_half(xs, ws_tiles, Wg4, Wu4, Wd, gid, act, prev, *, k, TM, P):
    H = xs.shape[1]
    I2 = Wg4.shape[2]
    I = Wd.shape[2]
    ntiles = P // TM
    grid_spec = pltpu.PrefetchScalarGridSpec(
        num_scalar_prefetch=2,
        grid=(ntiles,),
        in_specs=[
            pl.BlockSpec((TM, H), lambda t, g_r, a_r: (t, 0)),
            pl.BlockSpec((1, TM, 1), lambda t, g_r, a_r: (t, 0, 0)),
            pl.BlockSpec((1, 1, I2, H), lambda t, g_r, a_r: (g_r[t], k, 0, 0)),
            pl.BlockSpec((1, 1, I2, H), lambda t, g_r, a_r: (g_r[t], k, 0, 0)),
            pl.BlockSpec((1, H, I), lambda t, g_r, a_r: (g_r[t], 0, 0)),
            pl.BlockSpec((TM, H), lambda t, g_r, a_r: (t, 0)),
        ],
        out_specs=pl.BlockSpec((TM, H), lambda t, g_r, a_r: (t, 0)),
    )
    return pl.pallas_call(
        functools.partial(_grouped_ffn_kernel, first=(k == 0), k=k, I2=I2),
        grid_spec=grid_spec,
        out_shape=jax.ShapeDtypeStruct((P, H), jnp.float32),
    )(gid, act, xs, ws_tiles, Wg4, Wu4, Wd, prev)


def kernel(x, Wgate, Wg, Wu, Wd, Wg_s, Wu_s, Wd_s):
    bsz, seq_len, H = x.shape
    E, I, _ = Wg.shape
    T = bsz * seq_len
    TM = 128
    I2 = I // 2

    xf = x.reshape(T, H)

    logits = xf @ Wgate.T
    scores = jax.nn.softmax(logits, axis=-1)
    topk_w, topk_idx = jax.lax.top_k(scores, TOP_K)
    denom = jnp.sum(topk_w, axis=-1, keepdims=True) + 1e-20
    topk_w = topk_w / denom

    R = T * TOP_K
    e_pairs = topk_idx.reshape(R)
    w_pairs = topk_w.reshape(R)
    tok_pairs = jax.lax.broadcasted_iota(jnp.int32, (T, TOP_K), 0).reshape(R)

    oh = (e_pairs[:, None] == jnp.arange(E, dtype=e_pairs.dtype)[None, :])
    csum = jnp.cumsum(oh.astype(jnp.int32), axis=0)
    counts = csum[-1]
    rank = jnp.take_along_axis(csum, e_pairs[:, None].astype(jnp.int32),
                               axis=1)[:, 0] - 1
    pcounts = ((counts + TM - 1) // TM) * TM
    pcum = jnp.cumsum(pcounts)
    pstarts = pcum - pcounts
    dst = pstarts[e_pairs] + rank

    P = ((R + E * (TM - 1)) + 255) // 256 * 256
    ntiles = P // TM

    gather_src = jnp.zeros((P,), jnp.int32).at[dst].set(tok_pairs)
    ws_pad = jnp.zeros((P,), jnp.float32).at[dst].set(w_pairs)
    ws_tiles = ws_pad.reshape(ntiles, TM, 1)

    tile_base = jnp.arange(ntiles, dtype=jnp.int32) * TM
    gid = jnp.searchsorted(pcum, tile_base, side='right').astype(jnp.int32)
    act = (tile_base < pcum[-1]).astype(jnp.int32)
    gid = jnp.clip(gid, 0, E - 1)

    xs = jnp.take(xf, gather_src, axis=0)

    Wg4 = Wg.reshape(E, 2, I2, H)
    Wu4 = Wu.reshape(E, 2, I2, H)
    ys0 = _routed_half(xs, ws_tiles, Wg4, Wu4, Wd, gid, act,
                       jnp.zeros((P, H), jnp.float32), k=0, TM=TM, P=P)
    ys = _routed_half(xs, ws_tiles, Wg4, Wu4, Wd, gid, act,
                      ys0, k=1, TM=TM, P=P)

    TMS = 512
    y_sh = pl.pallas_call(
        _shared_ffn_kernel,
        grid=(T // TMS,),
        in_specs=[
            pl.BlockSpec((TMS, H), lambda t: (t, 0)),
            pl.BlockSpec((I, H), lambda t: (0, 0)),
            pl.BlockSpec((I, H), lambda t: (0, 0)),
            pl.BlockSpec((H, I), lambda t: (0, 0)),
        ],
        out_specs=pl.BlockSpec((TMS, H), lambda t: (t, 0)),
        out_shape=jax.ShapeDtypeStruct((T, H), jnp.float32),
    )(xf, Wg_s.astype(jnp.bfloat16), Wu_s.astype(jnp.bfloat16),
      Wd_s.astype(jnp.bfloat16))

    pos = dst.reshape(T, TOP_K)
    y = jnp.take(ys, pos[:, 0], axis=0) + jnp.take(ys, pos[:, 1], axis=0) + y_sh
    return y.reshape(bsz, seq_len, H)


# speedup vs baseline: 1.1316x; 1.1114x over previous
import functools

import jax
import jax.numpy as jnp
from jax.experimental import pallas as pl
from jax.experimental.pallas import tpu as pltpu

TOP_K = 2


def _gate_up_kernel(gid_ref, act_ref, xs_ref, ws_ref, wg_ref, wu_ref, h_ref):
    t = pl.program_id(0)

    @pl.when(act_ref[t] == 1)
    def _():
        x = xs_ref[...]
        g = jax.lax.dot_general(x, wg_ref[0], (((1,), (1,)), ((), ())),
                                preferred_element_type=jnp.float32)
        u = jax.lax.dot_general(x, wu_ref[0], (((1,), (1,)), ((), ())),
                                preferred_element_type=jnp.float32)
        h = g * jax.nn.sigmoid(g) * u * ws_ref[0]
        h_ref[...] = h.astype(jnp.bfloat16)


def _down_kernel(gid_ref, act_ref, h_ref, wd_ref, out_ref):
    t = pl.program_id(0)

    @pl.when(act_ref[t] == 1)
    def _():
        h = h_ref[...]
        wd = wd_ref[0].astype(jnp.bfloat16)
        out_ref[...] = jax.lax.dot_general(
            h, wd, (((1,), (1,)), ((), ())),
            preferred_element_type=jnp.float32)


def _shared_ffn_kernel(x_ref, wg_ref, wu_ref, wd_ref, out_ref):
    x = x_ref[...].astype(jnp.bfloat16)
    g = jax.lax.dot_general(x, wg_ref[...], (((1,), (1,)), ((), ())),
                            preferred_element_type=jnp.float32)
    u = jax.lax.dot_general(x, wu_ref[...], (((1,), (1,)), ((), ())),
                            preferred_element_type=jnp.float32)
    h = (g * jax.nn.sigmoid(g) * u).astype(jnp.bfloat16)
    out_ref[...] = jax.lax.dot_general(h, wd_ref[...], (((1,), (1,)), ((), ())),
                                       preferred_element_type=jnp.float32)


def kernel(x, Wgate, Wg, Wu, Wd, Wg_s, Wu_s, Wd_s):
    bsz, seq_len, H = x.shape
    E, I, _ = Wg.shape
    T = bsz * seq_len
    TM = 128

    xf = x.reshape(T, H)

    logits = xf @ Wgate.T
    scores = jax.nn.softmax(logits, axis=-1)
    eiota = jnp.arange(E, dtype=jnp.int32)[None, :]
    m1 = jnp.max(scores, axis=-1)
    i1 = jnp.argmax(scores, axis=-1).astype(jnp.int32)
    s2 = jnp.where(eiota == i1[:, None], -jnp.inf, scores)
    m2 = jnp.max(s2, axis=-1)
    i2 = jnp.argmax(s2, axis=-1).astype(jnp.int32)
    denom = m1 + m2 + 1e-20
    topk_w = jnp.stack([m1 / denom, m2 / denom], axis=-1)
    topk_idx = jnp.stack([i1, i2], axis=-1)

    R = T * TOP_K
    e_pairs = topk_idx.reshape(R)
    w_pairs = topk_w.reshape(R)
    tok_pairs = jax.lax.broadcasted_iota(jnp.int32, (T, TOP_K), 0).reshape(R)

    oh = (e_pairs[:, None] == eiota)
    csum = jnp.cumsum(oh.astype(jnp.int32), axis=0)
    counts = csum[-1]
    rank = jnp.take_along_axis(csum, e_pairs[:, None], axis=1)[:, 0] - 1
    pcounts = ((counts + TM - 1) // TM) * TM
    pcum = jnp.cumsum(pcounts)
    pstarts = pcum - pcounts
    dst = pstarts[e_pairs] + rank

    P = ((R + E * (TM - 1)) + 255) // 256 * 256
    ntiles = P // TM

    gather_src = jnp.zeros((P,), jnp.int32).at[dst].set(tok_pairs)
    ws_pad = jnp.zeros((P,), jnp.float32).at[dst].set(w_pairs)
    ws_tiles = ws_pad.reshape(ntiles, TM, 1)

    tile_base = jnp.arange(ntiles, dtype=jnp.int32) * TM
    gid = jnp.sum((tile_base[:, None] >= pcum[None, :]).astype(jnp.int32),
                  axis=1)
    act = (tile_base < pcum[-1]).astype(jnp.int32)
    gid = jnp.clip(gid, 0, E - 1).astype(jnp.int32)

    xs = jnp.take(xf, gather_src, axis=0)

    hmat = pl.pallas_call(
        _gate_up_kernel,
        grid_spec=pltpu.PrefetchScalarGridSpec(
            num_scalar_prefetch=2,
            grid=(ntiles,),
            in_specs=[
                pl.BlockSpec((TM, H), lambda t, g_r, a_r: (t, 0)),
                pl.BlockSpec((1, TM, 1), lambda t, g_r, a_r: (t, 0, 0)),
                pl.BlockSpec((1, I, H), lambda t, g_r, a_r: (g_r[t], 0, 0)),
                pl.BlockSpec((1, I, H), lambda t, g_r, a_r: (g_r[t], 0, 0)),
            ],
            out_specs=pl.BlockSpec((TM, I), lambda t, g_r, a_r: (t, 0)),
        ),
        out_shape=jax.ShapeDtypeStruct((P, I), jnp.bfloat16),
    )(gid, act, xs, ws_tiles, Wg, Wu)

    ys = pl.pallas_call(
        _down_kernel,
        grid_spec=pltpu.PrefetchScalarGridSpec(
            num_scalar_prefetch=2,
            grid=(ntiles,),
            in_specs=[
                pl.BlockSpec((TM, I), lambda t, g_r, a_r: (t, 0)),
                pl.BlockSpec((1, H, I), lambda t, g_r, a_r: (g_r[t], 0, 0)),
            ],
            out_specs=pl.BlockSpec((TM, H), lambda t, g_r, a_r: (t, 0)),
        ),
        out_shape=jax.ShapeDtypeStruct((P, H), jnp.float32),
    )(gid, act, hmat, Wd)

    TMS = 512
    y_sh = pl.pallas_call(
        _shared_ffn_kernel,
        grid=(T // TMS,),
        in_specs=[
            pl.BlockSpec((TMS, H), lambda t: (t, 0)),
            pl.BlockSpec((I, H), lambda t: (0, 0)),
            pl.BlockSpec((I, H), lambda t: (0, 0)),
            pl.BlockSpec((H, I), lambda t: (0, 0)),
        ],
        out_specs=pl.BlockSpec((TMS, H), lambda t: (t, 0)),
        out_shape=jax.ShapeDtypeStruct((T, H), jnp.float32),
    )(xf, Wg_s.astype(jnp.bfloat16), Wu_s.astype(jnp.bfloat16),
      Wd_s.astype(jnp.bfloat16))

    pos = dst.reshape(T, TOP_K)
    y = jnp.take(ys, pos[:, 0], axis=0) + jnp.take(ys, pos[:, 1], axis=0) + y_sh
    return y.reshape(bsz, seq_len, H)
